# baseline (speedup 1.0000x reference)
import functools

import jax
import jax.numpy as jnp
from jax import lax
from jax.experimental import pallas as pl
from jax.experimental.pallas import tpu as pltpu
from jax.experimental.pallas import tpu_sc as plsc

_NC = 2
_NS = 16
_G = 64
_EPS = 1e-5


def _chunk(ept: int, cap: int = 128) -> int:
    for k in range(cap, 0, -8):
        if ept % k == 0:
            return k
    raise ValueError(ept)


@functools.lru_cache(maxsize=None)
def _make_deg(n: int, e: int):
    nw = _NC * _NS
    ept = e // nw
    k = _chunk(ept)
    nch = ept // k
    rpt = (n // (8 * _NS)) * 8
    tail = n - rpt * _NS
    mesh = plsc.VectorSubcoreMesh(
        core_axis_name="c", subcore_axis_name="s", num_cores=_NC)
    out_type = [jax.ShapeDtypeStruct((n, 128), jnp.float32)] * 2
    scratch = [
        pltpu.VMEM((nch, k), jnp.int32),
        pltpu.VMEM((k, 128), jnp.float32),
        pltpu.VMEM_SHARED((n, 128), jnp.float32),
    ]

    @functools.partial(pl.kernel, mesh=mesh, out_type=out_type,
                       scratch_types=scratch)
    def deg(dsts, z128, o128, d0, d1, idst, ones_v, dacc):
        c = lax.axis_index("c")
        s = lax.axis_index("s")
        row0 = s * rpt
        pltpu.sync_copy(z128, dacc.at[pl.ds(row0, rpt)])
        if tail:
            @pl.when(s == _NS - 1)
            def _zt():
                pltpu.sync_copy(z128.at[pl.ds(0, tail)],
                                dacc.at[pl.ds(rpt * _NS, tail)])
        pltpu.sync_copy(o128, ones_v)
        pltpu.sync_copy(dsts.at[c * _NS + s], idst)
        plsc.subcore_barrier()

        def body(t, carry):
            pltpu.sync_copy(ones_v, dacc.at[idst.at[t]], add=True)
            return carry
        lax.fori_loop(0, nch, body, 0)
        plsc.subcore_barrier()

        sl = pl.ds(row0, rpt)
        tl = pl.ds(rpt * _NS, tail) if tail else None

        @pl.when(c == 0)
        def _o0():
            pltpu.sync_copy(dacc.at[sl], d0.at[sl])
            if tail:
                @pl.when(s == _NS - 1)
                def _o0t():
                    pltpu.sync_copy(dacc.at[tl], d0.at[tl])

        @pl.when(c == 1)
        def _o1():
            pltpu.sync_copy(dacc.at[sl], d1.at[sl])
            if tail:
                @pl.when(s == _NS - 1)
                def _o1t():
                    pltpu.sync_copy(dacc.at[tl], d1.at[tl])

    return deg


def _deg_call(dstd, n):
    e = dstd.shape[0] * dstd.shape[1] * dstd.shape[2]
    rpt = (n // (8 * _NS)) * 8
    k = dstd.shape[2]
    z128 = jnp.zeros((rpt, 128), jnp.float32)
    o128 = jnp.ones((k, 128), jnp.float32)
    return _make_deg(n, e)(dstd, z128, o128)


@functools.lru_cache(maxsize=None)
def _make_aggr(n: int, cpt: int, ngrp: int):
    k = 128
    cpg = cpt // ngrp
    rpt = (n // (8 * _NS)) * 8
    tail = n - rpt * _NS
    mesh = plsc.VectorSubcoreMesh(
        core_axis_name="c", subcore_axis_name="s", num_cores=_NC)

    out_type = [
        jax.ShapeDtypeStruct((n, 128), jnp.float32),
        jax.ShapeDtypeStruct((n, 128), jnp.float32),
    ]
    scratch = [
        pltpu.VMEM((cpg, k), jnp.int32),
        pltpu.VMEM((cpg, k), jnp.int32),
        pltpu.VMEM((k, 128), jnp.float32),
        pltpu.VMEM((k, 128), jnp.float32),
        pltpu.VMEM_SHARED((n + 8, 128), jnp.float32),
        pltpu.SemaphoreType.DMA,
        pltpu.SemaphoreType.DMA,
    ]

    @functools.partial(pl.kernel, mesh=mesh, out_type=out_type,
                       scratch_types=scratch)
    def aggr(ya, yb, srcs, dsts, z128, aa, ab,
             isrc, idst, rows0, rows1, acc, sem0, sem1):
        c = lax.axis_index("c")
        s = lax.axis_index("s")
        row0 = s * rpt

        pltpu.sync_copy(z128, acc.at[pl.ds(row0, rpt)])
        if tail:
            @pl.when(s == _NS - 1)
            def _zero_tail():
                pltpu.sync_copy(z128.at[pl.ds(0, tail + 8)],
                                acc.at[pl.ds(rpt * _NS, tail + 8)])
        plsc.subcore_barrier()

        def run(yref):
            def gwait(t, buf, sem):
                pltpu.make_async_copy(yref.at[isrc.at[t]], buf, sem).wait()

            def scat(t, buf):
                pltpu.sync_copy(buf, acc.at[idst.at[t]], add=True)

            def group(g, carry):
                pltpu.sync_copy(srcs.at[s, g], isrc)
                pltpu.sync_copy(dsts.at[s, g], idst)
                pltpu.async_copy(yref.at[isrc.at[0]], rows0, sem0)

                def pair(i, carry):
                    t0 = 2 * i
                    pltpu.async_copy(yref.at[isrc.at[t0 + 1]], rows1, sem1)
                    gwait(t0, rows0, sem0)
                    scat(t0, rows0)

                    @pl.when(t0 + 2 < cpg)
                    def _next():
                        pltpu.async_copy(yref.at[isrc.at[t0 + 2]], rows0,
                                         sem0)
                    gwait(t0 + 1, rows1, sem1)
                    scat(t0 + 1, rows1)
                    return carry
                lax.fori_loop(0, cpg // 2, pair, 0)
                if cpg % 2:
                    gwait(cpg - 1, rows0, sem0)
                    scat(cpg - 1, rows0)
                return carry
            lax.fori_loop(0, ngrp, group, 0)

        @pl.when(c == 0)
        def _run_a():
            run(ya)

        @pl.when(c == 1)
        def _run_b():
            run(yb)

        plsc.subcore_barrier()

        sl = pl.ds(row0, rpt)
        tl = pl.ds(rpt * _NS, tail) if tail else None

        @pl.when(c == 0)
        def _out_a():
            pltpu.sync_copy(acc.at[sl], aa.at[sl])
            if tail:
                @pl.when(s == _NS - 1)
                def _out_a_tail():
                    pltpu.sync_copy(acc.at[tl], aa.at[tl])

        @pl.when(c == 1)
        def _out_b():
            pltpu.sync_copy(acc.at[sl], ab.at[sl])
            if tail:
                @pl.when(s == _NS - 1)
                def _out_b_tail():
                    pltpu.sync_copy(acc.at[tl], ab.at[tl])

    return aggr


def _aggr_plan(e):
    ngrp = 4
    cpt = -(-e // (_NS * 128))
    cpt += (-cpt) % ngrp
    return cpt, ngrp, cpt * 128 * _NS - e


def _aggr_call(ya, yb, srcs, dsts):
    n = ya.shape[0]
    ngrp, cpg = srcs.shape[1], srcs.shape[2]
    rpt = (n // (8 * _NS)) * 8
    tail = n - rpt * _NS
    z128 = jnp.zeros((max(rpt, tail + 8), 128), jnp.float32)
    return _make_aggr(n, ngrp * cpg, ngrp)(ya, yb, srcs, dsts, z128)


@functools.lru_cache(maxsize=None)
def _make_layer(n: int, affine: bool, pool: bool, bn: int):
    grid = n // bn
    nf = float(n)

    def body(*refs):
        i = pl.program_id(0)
        it = iter(refs)
        ya, yb, aa, ab = next(it), next(it), next(it), next(it)
        if affine:
            d0, d1, s_in, q_in, g_in, be_in = (next(it), next(it), next(it),
                                               next(it), next(it), next(it))
        wa, ba, wb, bb = next(it), next(it), next(it), next(it)
        if pool:
            bat = next(it)
        oa, ob, s_o, q_o = next(it), next(it), next(it), next(it)
        if pool:
            p_o, cnt_o = next(it), next(it)

        y = jnp.concatenate([ya[...], yb[...]], axis=1)
        agg = jnp.concatenate([aa[...], ab[...]], axis=1)
        if affine:
            mu = s_in[...] * (1.0 / nf)
            var = q_in[...] * (1.0 / nf) - mu * mu
            a = g_in[...] * lax.rsqrt(var + _EPS)
            cbias = be_in[...] - mu * a
            m = a * (y + agg) + cbias * (
                1.0 + (d0[...] + d1[...])[:, 0:1])
        else:
            m = y + agg
        t = jnp.maximum(
            jnp.dot(m, wa[...], preferred_element_type=jnp.float32) + ba[...],
            0.0)
        y2 = jnp.maximum(
            jnp.dot(t, wb[...], preferred_element_type=jnp.float32) + bb[...],
            0.0)
        oa[...] = y2[:, :128]
        ob[...] = y2[:, 128:]
        s_p = jnp.sum(y2, axis=0, keepdims=True)
        q_p = jnp.sum(y2 * y2, axis=0, keepdims=True)
        if pool:
            iota_g = lax.broadcasted_iota(jnp.int32, (_G, bn), 0)
            onehot = (jnp.reshape(bat[...], (1, bn)) == iota_g
                      ).astype(jnp.float32)
            p_p = jnp.dot(onehot, y2, preferred_element_type=jnp.float32)
            c_p = jnp.sum(onehot, axis=1, keepdims=True) + jnp.zeros(
                (_G, 128), jnp.float32)

        @pl.when(i == 0)
        def _init():
            s_o[...] = s_p
            q_o[...] = q_p
            if pool:
                p_o[...] = p_p
                cnt_o[...] = c_p

        @pl.when(i > 0)
        def _accum():
            s_o[...] += s_p
            q_o[...] += q_p
            if pool:
                p_o[...] += p_p
                cnt_o[...] += c_p

    half = pl.BlockSpec((bn, 128), lambda i: (i, 0))
    full = pl.BlockSpec((256, 256), lambda i: (0, 0))
    vec = pl.BlockSpec((1, 256), lambda i: (0, 0))
    in_specs = [half, half, half, half]
    if affine:
        in_specs += [pl.BlockSpec((bn, 128), lambda i: (i, 0)),
                     pl.BlockSpec((bn, 128), lambda i: (i, 0)),
                     vec, vec, vec, vec]
    in_specs += [full, vec, full, vec]
    if pool:
        in_specs += [pl.BlockSpec((1, 1, bn), lambda i: (i, 0, 0))]
    out_specs = [half, half, vec, vec]
    out_shape = [jax.ShapeDtypeStruct((n, 128), jnp.float32),
                 jax.ShapeDtypeStruct((n, 128), jnp.float32),
                 jax.ShapeDtypeStruct((1, 256), jnp.float32),
                 jax.ShapeDtypeStruct((1, 256), jnp.float32)]
    if pool:
        out_specs += [pl.BlockSpec((_G, 256), lambda i: (0, 0)),
                      pl.BlockSpec((_G, 128), lambda i: (0, 0))]
        out_shape += [jax.ShapeDtypeStruct((_G, 256), jnp.float32),
                      jax.ShapeDtypeStruct((_G, 128), jnp.float32)]

    return pl.pallas_call(
        body, grid=(grid,), in_specs=in_specs, out_specs=out_specs,
        out_shape=out_shape)


@functools.lru_cache(maxsize=None)
def _make_head(n: int, out_dim: int):
    nf = float(n)

    def body(p, cnt, s_in, q_in, g_in, be_in, wf1, bf1, wf2, bf2, out):
        mu = s_in[...] * (1.0 / nf)
        var = q_in[...] * (1.0 / nf) - mu * mu
        a = g_in[...] * lax.rsqrt(var + _EPS)
        cbias = be_in[...] - mu * a
        pooled = a * p[...] + cbias * cnt[...][:, 0:1]
        t = jnp.maximum(
            jnp.dot(pooled, wf1[...], preferred_element_type=jnp.float32)
            + bf1[...], 0.0)
        logits = jnp.dot(t, wf2[...], preferred_element_type=jnp.float32) \
            + bf2[...]
        mx = jnp.max(logits, axis=-1, keepdims=True)
        z = logits - mx
        out[...] = z - jnp.log(jnp.sum(jnp.exp(z), axis=-1, keepdims=True))

    return pl.pallas_call(
        body, out_shape=jax.ShapeDtypeStruct((_G, out_dim), jnp.float32))


def kernel(x, edge_index, batch, W1a, b1a, W1b, b1b, g1, be1, W2a, b2a, W2b,
           b2b, g2, be2, W3a, b3a, W3b, b3b, g3, be3, Wf1, bf1, Wf2, bf2):
    n, d = x.shape
    e = edge_index.shape[1]
    out_dim = Wf2.shape[1]
    bn = 1000
    cpt, ngrp, pad = _aggr_plan(e)

    kd = _chunk(e // (_NC * _NS))
    xa, xb = x[:, :128], x[:, 128:]
    srcp = jnp.concatenate(
        [edge_index[0], jnp.zeros((pad,), jnp.int32)])
    dstp = jnp.concatenate(
        [edge_index[1], jnp.full((pad,), n, jnp.int32)])
    srcs = srcp.reshape(_NS, ngrp, cpt // ngrp, 128)
    dsts = dstp.reshape(_NS, ngrp, cpt // ngrp, 128)
    dstd = edge_index[1].reshape(_NC * _NS, e // (_NC * _NS) // kd, kd)
    bat3 = batch.reshape(n // bn, 1, bn)

    def r(v):
        return v.reshape(1, -1)

    deg0, deg1 = _deg_call(dstd, n)
    a1a, a1b = _aggr_call(xa, xb, srcs, dsts)
    y1a, y1b, s1, q1 = _make_layer(n, False, False, bn)(
        xa, xb, a1a, a1b, W1a, r(b1a), W1b, r(b1b))

    a2a, a2b = _aggr_call(y1a, y1b, srcs, dsts)
    y2a, y2b, s2, q2 = _make_layer(n, True, False, bn)(
        y1a, y1b, a2a, a2b, deg0, deg1, s1, q1, r(g1), r(be1),
        W2a, r(b2a), W2b, r(b2b))

    a3a, a3b = _aggr_call(y2a, y2b, srcs, dsts)
    _, _, s3, q3, p, cnt = _make_layer(n, True, True, bn)(
        y2a, y2b, a3a, a3b, deg0, deg1, s2, q2, r(g2), r(be2),
        W3a, r(b3a), W3b, r(b3b), bat3)

    return _make_head(n, out_dim)(
        p, cnt, s3, q3, r(g3), r(be3), Wf1, r(bf1), Wf2, r(bf2))


# speedup vs baseline: 1.0732x; 1.0732x over previous
import functools

import jax
import jax.numpy as jnp
from jax import lax
from jax.experimental import pallas as pl
from jax.experimental.pallas import tpu as pltpu
from jax.experimental.pallas import tpu_sc as plsc

_NC = 2
_NS = 16
_G = 64
_EPS = 1e-5


def _chunk(ept: int, cap: int = 128) -> int:
    for k in range(cap, 0, -8):
        if ept % k == 0:
            return k
    raise ValueError(ept)


@functools.lru_cache(maxsize=None)
def _make_deg(n: int, e: int):
    nw = _NC * _NS
    ept = e // nw
    k = _chunk(ept)
    nch = ept // k
    rpt = (n // (8 * _NS)) * 8
    tail = n - rpt * _NS
    mesh = plsc.VectorSubcoreMesh(
        core_axis_name="c", subcore_axis_name="s", num_cores=_NC)
    out_type = [jax.ShapeDtypeStruct((n, 128), jnp.float32)] * 2
    scratch = [
        pltpu.VMEM((nch, k), jnp.int32),
        pltpu.VMEM((k, 128), jnp.float32),
        pltpu.VMEM_SHARED((n, 128), jnp.float32),
    ]

    @functools.partial(pl.kernel, mesh=mesh, out_type=out_type,
                       scratch_types=scratch)
    def deg(dsts, z128, o128, d0, d1, idst, ones_v, dacc):
        c = lax.axis_index("c")
        s = lax.axis_index("s")
        row0 = s * rpt
        pltpu.sync_copy(z128, dacc.at[pl.ds(row0, rpt)])
        if tail:
            @pl.when(s == _NS - 1)
            def _zt():
                pltpu.sync_copy(z128.at[pl.ds(0, tail)],
                                dacc.at[pl.ds(rpt * _NS, tail)])
        pltpu.sync_copy(o128, ones_v)
        pltpu.sync_copy(dsts.at[c * _NS + s], idst)
        plsc.subcore_barrier()

        def body(t, carry):
            pltpu.sync_copy(ones_v, dacc.at[idst.at[t]], add=True)
            return carry
        lax.fori_loop(0, nch, body, 0)
        plsc.subcore_barrier()

        sl = pl.ds(row0, rpt)
        tl = pl.ds(rpt * _NS, tail) if tail else None

        @pl.when(c == 0)
        def _o0():
            pltpu.sync_copy(dacc.at[sl], d0.at[sl])
            if tail:
                @pl.when(s == _NS - 1)
                def _o0t():
                    pltpu.sync_copy(dacc.at[tl], d0.at[tl])

        @pl.when(c == 1)
        def _o1():
            pltpu.sync_copy(dacc.at[sl], d1.at[sl])
            if tail:
                @pl.when(s == _NS - 1)
                def _o1t():
                    pltpu.sync_copy(dacc.at[tl], d1.at[tl])

    return deg


def _deg_call(dstd, n):
    e = dstd.shape[0] * dstd.shape[1] * dstd.shape[2]
    rpt = (n // (8 * _NS)) * 8
    k = dstd.shape[2]
    z128 = jnp.zeros((rpt, 128), jnp.float32)
    o128 = jnp.ones((k, 128), jnp.float32)
    return _make_deg(n, e)(dstd, z128, o128)


@functools.lru_cache(maxsize=None)
def _make_aggr(n: int, cpt: int, ngrp: int):
    k = 128
    cpg = cpt // ngrp
    rpt = (n // (8 * _NS)) * 8
    tail = n - rpt * _NS
    mesh = plsc.VectorSubcoreMesh(
        core_axis_name="c", subcore_axis_name="s", num_cores=_NC)

    out_type = [
        jax.ShapeDtypeStruct((n, 128), jnp.float32),
        jax.ShapeDtypeStruct((n, 128), jnp.float32),
    ]
    scratch = [
        pltpu.VMEM((cpg, k), jnp.int32),
        pltpu.VMEM((cpg, k), jnp.int32),
        pltpu.VMEM((k, 128), jnp.float32),
        pltpu.VMEM((k, 128), jnp.float32),
        pltpu.VMEM_SHARED((n + 8, 128), jnp.float32),
        pltpu.SemaphoreType.DMA,
        pltpu.SemaphoreType.DMA,
    ]

    @functools.partial(pl.kernel, mesh=mesh, out_type=out_type,
                       scratch_types=scratch)
    def aggr(ya, yb, srcs, dsts, z128, aa, ab,
             isrc, idst, rows0, rows1, acc, sem0, sem1):
        c = lax.axis_index("c")
        s = lax.axis_index("s")
        row0 = s * rpt

        pltpu.sync_copy(z128, acc.at[pl.ds(row0, rpt)])
        if tail:
            @pl.when(s == _NS - 1)
            def _zero_tail():
                pltpu.sync_copy(z128.at[pl.ds(0, tail + 8)],
                                acc.at[pl.ds(rpt * _NS, tail + 8)])
        plsc.subcore_barrier()

        def run(yref):
            def group(g, carry):
                pltpu.sync_copy(srcs.at[s, g], isrc)
                pltpu.sync_copy(dsts.at[s, g], idst)

                def body(t, carry):
                    pltpu.async_copy(yref.at[isrc.at[t]], rows0, sem0).wait()
                    pltpu.sync_copy(rows0, acc.at[idst.at[t]], add=True)
                    return carry
                lax.fori_loop(0, cpg, body, 0)
                return carry
            lax.fori_loop(0, ngrp, group, 0)

        @pl.when(c == 0)
        def _run_a():
            run(ya)

        @pl.when(c == 1)
        def _run_b():
            run(yb)

        plsc.subcore_barrier()

        sl = pl.ds(row0, rpt)
        tl = pl.ds(rpt * _NS, tail) if tail else None

        @pl.when(c == 0)
        def _out_a():
            pltpu.sync_copy(acc.at[sl], aa.at[sl])
            if tail:
                @pl.when(s == _NS - 1)
                def _out_a_tail():
                    pltpu.sync_copy(acc.at[tl], aa.at[tl])

        @pl.when(c == 1)
        def _out_b():
            pltpu.sync_copy(acc.at[sl], ab.at[sl])
            if tail:
                @pl.when(s == _NS - 1)
                def _out_b_tail():
                    pltpu.sync_copy(acc.at[tl], ab.at[tl])

    return aggr


def _aggr_plan(e):
    ngrp = 1
    cpt = -(-e // (_NS * 128))
    cpt += (-cpt) % ngrp
    return cpt, ngrp, cpt * 128 * _NS - e


def _aggr_call(ya, yb, srcs, dsts):
    n = ya.shape[0]
    ngrp, cpg = srcs.shape[1], srcs.shape[2]
    rpt = (n // (8 * _NS)) * 8
    tail = n - rpt * _NS
    z128 = jnp.zeros((max(rpt, tail + 8), 128), jnp.float32)
    return _make_aggr(n, ngrp * cpg, ngrp)(ya, yb, srcs, dsts, z128)


@functools.lru_cache(maxsize=None)
def _make_layer(n: int, affine: bool, pool: bool, bn: int):
    grid = n // bn
    nf = float(n)

    def body(*refs):
        i = pl.program_id(0)
        it = iter(refs)
        ya, yb, aa, ab = next(it), next(it), next(it), next(it)
        if affine:
            d0, d1, s_in, q_in, g_in, be_in = (next(it), next(it), next(it),
                                               next(it), next(it), next(it))
        wa, ba, wb, bb = next(it), next(it), next(it), next(it)
        if pool:
            bat = next(it)
        oa, ob, s_o, q_o = next(it), next(it), next(it), next(it)
        if pool:
            p_o, cnt_o = next(it), next(it)

        y = jnp.concatenate([ya[...], yb[...]], axis=1)
        agg = jnp.concatenate([aa[...], ab[...]], axis=1)
        if affine:
            mu = s_in[...] * (1.0 / nf)
            var = q_in[...] * (1.0 / nf) - mu * mu
            a = g_in[...] * lax.rsqrt(var + _EPS)
            cbias = be_in[...] - mu * a
            m = a * (y + agg) + cbias * (
                1.0 + (d0[...] + d1[...])[:, 0:1])
        else:
            m = y + agg
        t = jnp.maximum(
            jnp.dot(m, wa[...], preferred_element_type=jnp.float32) + ba[...],
            0.0)
        y2 = jnp.maximum(
            jnp.dot(t, wb[...], preferred_element_type=jnp.float32) + bb[...],
            0.0)
        oa[...] = y2[:, :128]
        ob[...] = y2[:, 128:]
        s_p = jnp.sum(y2, axis=0, keepdims=True)
        q_p = jnp.sum(y2 * y2, axis=0, keepdims=True)
        if pool:
            iota_g = lax.broadcasted_iota(jnp.int32, (_G, bn), 0)
            onehot = (jnp.reshape(bat[...], (1, bn)) == iota_g
                      ).astype(jnp.float32)
            p_p = jnp.dot(onehot, y2, preferred_element_type=jnp.float32)
            c_p = jnp.sum(onehot, axis=1, keepdims=True) + jnp.zeros(
                (_G, 128), jnp.float32)

        @pl.when(i == 0)
        def _init():
            s_o[...] = s_p
            q_o[...] = q_p
            if pool:
                p_o[...] = p_p
                cnt_o[...] = c_p

        @pl.when(i > 0)
        def _accum():
            s_o[...] += s_p
            q_o[...] += q_p
            if pool:
                p_o[...] += p_p
                cnt_o[...] += c_p

    half = pl.BlockSpec((bn, 128), lambda i: (i, 0))
    full = pl.BlockSpec((256, 256), lambda i: (0, 0))
    vec = pl.BlockSpec((1, 256), lambda i: (0, 0))
    in_specs = [half, half, half, half]
    if affine:
        in_specs += [pl.BlockSpec((bn, 128), lambda i: (i, 0)),
                     pl.BlockSpec((bn, 128), lambda i: (i, 0)),
                     vec, vec, vec, vec]
    in_specs += [full, vec, full, vec]
    if pool:
        in_specs += [pl.BlockSpec((1, 1, bn), lambda i: (i, 0, 0))]
    out_specs = [half, half, vec, vec]
    out_shape = [jax.ShapeDtypeStruct((n, 128), jnp.float32),
                 jax.ShapeDtypeStruct((n, 128), jnp.float32),
                 jax.ShapeDtypeStruct((1, 256), jnp.float32),
                 jax.ShapeDtypeStruct((1, 256), jnp.float32)]
    if pool:
        out_specs += [pl.BlockSpec((_G, 256), lambda i: (0, 0)),
                      pl.BlockSpec((_G, 128), lambda i: (0, 0))]
        out_shape += [jax.ShapeDtypeStruct((_G, 256), jnp.float32),
                      jax.ShapeDtypeStruct((_G, 128), jnp.float32)]

    return pl.pallas_call(
        body, grid=(grid,), in_specs=in_specs, out_specs=out_specs,
        out_shape=out_shape)


@functools.lru_cache(maxsize=None)
def _make_head(n: int, out_dim: int):
    nf = float(n)

    def body(p, cnt, s_in, q_in, g_in, be_in, wf1, bf1, wf2, bf2, out):
        mu = s_in[...] * (1.0 / nf)
        var = q_in[...] * (1.0 / nf) - mu * mu
        a = g_in[...] * lax.rsqrt(var + _EPS)
        cbias = be_in[...] - mu * a
        pooled = a * p[...] + cbias * cnt[...][:, 0:1]
        t = jnp.maximum(
            jnp.dot(pooled, wf1[...], preferred_element_type=jnp.float32)
            + bf1[...], 0.0)
        logits = jnp.dot(t, wf2[...], preferred_element_type=jnp.float32) \
            + bf2[...]
        mx = jnp.max(logits, axis=-1, keepdims=True)
        z = logits - mx
        out[...] = z - jnp.log(jnp.sum(jnp.exp(z), axis=-1, keepdims=True))

    return pl.pallas_call(
        body, out_shape=jax.ShapeDtypeStruct((_G, out_dim), jnp.float32))


def kernel(x, edge_index, batch, W1a, b1a, W1b, b1b, g1, be1, W2a, b2a, W2b,
           b2b, g2, be2, W3a, b3a, W3b, b3b, g3, be3, Wf1, bf1, Wf2, bf2):
    n, d = x.shape
    e = edge_index.shape[1]
    out_dim = Wf2.shape[1]
    bn = 1000
    cpt, ngrp, pad = _aggr_plan(e)

    kd = _chunk(e // (_NC * _NS))
    xa, xb = x[:, :128], x[:, 128:]
    srcp = jnp.concatenate(
        [edge_index[0], jnp.zeros((pad,), jnp.int32)])
    dstp = jnp.concatenate(
        [edge_index[1], jnp.full((pad,), n, jnp.int32)])
    srcs = srcp.reshape(_NS, ngrp, cpt // ngrp, 128)
    dsts = dstp.reshape(_NS, ngrp, cpt // ngrp, 128)
    dstd = edge_index[1].reshape(_NC * _NS, e // (_NC * _NS) // kd, kd)
    bat3 = batch.reshape(n // bn, 1, bn)

    def r(v):
        return v.reshape(1, -1)

    deg0, deg1 = _deg_call(dstd, n)
    a1a, a1b = _aggr_call(xa, xb, srcs, dsts)
    y1a, y1b, s1, q1 = _make_layer(n, False, False, bn)(
        xa, xb, a1a, a1b, W1a, r(b1a), W1b, r(b1b))

    a2a, a2b = _aggr_call(y1a, y1b, srcs, dsts)
    y2a, y2b, s2, q2 = _make_layer(n, True, False, bn)(
        y1a, y1b, a2a, a2b, deg0, deg1, s1, q1, r(g1), r(be1),
        W2a, r(b2a), W2b, r(b2b))

    a3a, a3b = _aggr_call(y2a, y2b, srcs, dsts)
    _, _, s3, q3, p, cnt = _make_layer(n, True, True, bn)(
        y2a, y2b, a3a, a3b, deg0, deg1, s2, q2, r(g2), r(be2),
        W3a, r(b3a), W3b, r(b3b), bat3)

    return _make_head(n, out_dim)(
        p, cnt, s3, q3, r(g3), r(be3), Wf1, r(bf1), Wf2, r(bf2))


# speedup vs baseline: 1.2903x; 1.2023x over previous
import functools

import jax
import jax.numpy as jnp
from jax import lax
from jax.experimental import pallas as pl
from jax.experimental.pallas import tpu as pltpu
from jax.experimental.pallas import tpu_sc as plsc

_NC = 2
_NS = 16
_G = 64
_EPS = 1e-5


def _chunk(ept: int, cap: int = 128) -> int:
    for k in range(cap, 0, -8):
        if ept % k == 0:
            return k
    raise ValueError(ept)


@functools.lru_cache(maxsize=None)
def _make_deg(n: int, e: int):
    nw = _NC * _NS
    ept = e // nw
    k = _chunk(ept)
    nch = ept // k
    rpt = (n // (8 * _NS)) * 8
    tail = n - rpt * _NS
    mesh = plsc.VectorSubcoreMesh(
        core_axis_name="c", subcore_axis_name="s", num_cores=_NC)
    out_type = [jax.ShapeDtypeStruct((n, 128), jnp.float32)] * 2
    scratch = [
        pltpu.VMEM((nch, k), jnp.int32),
        pltpu.VMEM((k, 128), jnp.float32),
        pltpu.VMEM_SHARED((n, 128), jnp.float32),
    ]

    @functools.partial(pl.kernel, mesh=mesh, out_type=out_type,
                       scratch_types=scratch)
    def deg(dsts, z128, o128, d0, d1, idst, ones_v, dacc):
        c = lax.axis_index("c")
        s = lax.axis_index("s")
        row0 = s * rpt
        pltpu.sync_copy(z128, dacc.at[pl.ds(row0, rpt)])
        if tail:
            @pl.when(s == _NS - 1)
            def _zt():
                pltpu.sync_copy(z128.at[pl.ds(0, tail)],
                                dacc.at[pl.ds(rpt * _NS, tail)])
        pltpu.sync_copy(o128, ones_v)
        pltpu.sync_copy(dsts.at[c * _NS + s], idst)
        plsc.subcore_barrier()

        def body(t, carry):
            pltpu.sync_copy(ones_v, dacc.at[idst.at[t]], add=True)
            return carry
        lax.fori_loop(0, nch, body, 0)
        plsc.subcore_barrier()

        sl = pl.ds(row0, rpt)
        tl = pl.ds(rpt * _NS, tail) if tail else None

        @pl.when(c == 0)
        def _o0():
            pltpu.sync_copy(dacc.at[sl], d0.at[sl])
            if tail:
                @pl.when(s == _NS - 1)
                def _o0t():
                    pltpu.sync_copy(dacc.at[tl], d0.at[tl])

        @pl.when(c == 1)
        def _o1():
            pltpu.sync_copy(dacc.at[sl], d1.at[sl])
            if tail:
                @pl.when(s == _NS - 1)
                def _o1t():
                    pltpu.sync_copy(dacc.at[tl], d1.at[tl])

    return deg


def _deg_call(dstd, n):
    e = dstd.shape[0] * dstd.shape[1] * dstd.shape[2]
    rpt = (n // (8 * _NS)) * 8
    k = dstd.shape[2]
    z128 = jnp.zeros((rpt, 128), jnp.float32)
    o128 = jnp.ones((k, 128), jnp.float32)
    return _make_deg(n, e)(dstd, z128, o128)


@functools.lru_cache(maxsize=None)
def _make_aggr(n: int, e: int, cap: int):
    ept = e // _NS
    k = _chunk(ept, cap)
    nch = ept // k
    rpt = (n // (8 * _NS)) * 8
    tail = n - rpt * _NS
    mesh = plsc.VectorSubcoreMesh(
        core_axis_name="c", subcore_axis_name="s", num_cores=_NC)

    out_type = [
        jax.ShapeDtypeStruct((n, 128), jnp.float32),
        jax.ShapeDtypeStruct((n, 128), jnp.float32),
    ]
    scratch = [
        pltpu.VMEM((nch, k), jnp.int32),
        pltpu.VMEM((nch, k), jnp.int32),
        pltpu.VMEM((k, 128), jnp.float32),
        pltpu.VMEM_SHARED((n, 128), jnp.float32),
        pltpu.SemaphoreType.DMA,
    ]

    @functools.partial(pl.kernel, mesh=mesh, out_type=out_type,
                       scratch_types=scratch)
    def aggr(ya, yb, srcs, dsts, z128, aa, ab, isrc, idst, rows, acc, sem):
        c = lax.axis_index("c")
        s = lax.axis_index("s")
        row0 = s * rpt

        pltpu.sync_copy(z128, acc.at[pl.ds(row0, rpt)])
        if tail:
            @pl.when(s == _NS - 1)
            def _zero_tail():
                pltpu.sync_copy(z128.at[pl.ds(0, tail)],
                                acc.at[pl.ds(rpt * _NS, tail)])

        pltpu.sync_copy(srcs.at[s], isrc)
        pltpu.sync_copy(dsts.at[s], idst)
        plsc.subcore_barrier()

        def run(yref):
            def body(t, carry):
                pltpu.async_copy(yref.at[isrc.at[t]], rows, sem).wait()
                pltpu.sync_copy(rows, acc.at[idst.at[t]], add=True)
                return carry
            lax.fori_loop(0, nch, body, 0)

        @pl.when(c == 0)
        def _run_a():
            run(ya)

        @pl.when(c == 1)
        def _run_b():
            run(yb)

        plsc.subcore_barrier()

        sl = pl.ds(row0, rpt)
        tl = pl.ds(rpt * _NS, tail) if tail else None

        @pl.when(c == 0)
        def _out_a():
            pltpu.sync_copy(acc.at[sl], aa.at[sl])
            if tail:
                @pl.when(s == _NS - 1)
                def _out_a_tail():
                    pltpu.sync_copy(acc.at[tl], aa.at[tl])

        @pl.when(c == 1)
        def _out_b():
            pltpu.sync_copy(acc.at[sl], ab.at[sl])
            if tail:
                @pl.when(s == _NS - 1)
                def _out_b_tail():
                    pltpu.sync_copy(acc.at[tl], ab.at[tl])

    return aggr


_AGGR_CAP = 80


def _aggr_call(ya, yb, srcs, dsts):
    n = ya.shape[0]
    e = srcs.shape[0] * srcs.shape[1] * srcs.shape[2]
    rpt = (n // (8 * _NS)) * 8
    z128 = jnp.zeros((rpt, 128), jnp.float32)
    return _make_aggr(n, e, _AGGR_CAP)(ya, yb, srcs, dsts, z128)


@functools.lru_cache(maxsize=None)
def _make_layer(n: int, affine: bool, pool: bool, bn: int):
    grid = n // bn
    nf = float(n)

    def body(*refs):
        i = pl.program_id(0)
        it = iter(refs)
        ya, yb, aa, ab = next(it), next(it), next(it), next(it)
        if affine:
            d0, d1, s_in, q_in, g_in, be_in = (next(it), next(it), next(it),
                                               next(it), next(it), next(it))
        wa, ba, wb, bb = next(it), next(it), next(it), next(it)
        if pool:
            bat = next(it)
        oa, ob, s_o, q_o = next(it), next(it), next(it), next(it)
        if pool:
            p_o, cnt_o = next(it), next(it)

        y = jnp.concatenate([ya[...], yb[...]], axis=1)
        agg = jnp.concatenate([aa[...], ab[...]], axis=1)
        if affine:
            mu = s_in[...] * (1.0 / nf)
            var = q_in[...] * (1.0 / nf) - mu * mu
            a = g_in[...] * lax.rsqrt(var + _EPS)
            cbias = be_in[...] - mu * a
            m = a * (y + agg) + cbias * (
                1.0 + (d0[...] + d1[...])[:, 0:1])
        else:
            m = y + agg
        t = jnp.maximum(
            jnp.dot(m, wa[...], preferred_element_type=jnp.float32) + ba[...],
            0.0)
        y2 = jnp.maximum(
            jnp.dot(t, wb[...], preferred_element_type=jnp.float32) + bb[...],
            0.0)
        oa[...] = y2[:, :128]
        ob[...] = y2[:, 128:]
        s_p = jnp.sum(y2, axis=0, keepdims=True)
        q_p = jnp.sum(y2 * y2, axis=0, keepdims=True)
        if pool:
            iota_g = lax.broadcasted_iota(jnp.int32, (_G, bn), 0)
            onehot = (jnp.reshape(bat[...], (1, bn)) == iota_g
                      ).astype(jnp.float32)
            p_p = jnp.dot(onehot, y2, preferred_element_type=jnp.float32)
            c_p = jnp.sum(onehot, axis=1, keepdims=True) + jnp.zeros(
                (_G, 128), jnp.float32)

        @pl.when(i == 0)
        def _init():
            s_o[...] = s_p
            q_o[...] = q_p
            if pool:
                p_o[...] = p_p
                cnt_o[...] = c_p

        @pl.when(i > 0)
        def _accum():
            s_o[...] += s_p
            q_o[...] += q_p
            if pool:
                p_o[...] += p_p
                cnt_o[...] += c_p

    half = pl.BlockSpec((bn, 128), lambda i: (i, 0))
    full = pl.BlockSpec((256, 256), lambda i: (0, 0))
    vec = pl.BlockSpec((1, 256), lambda i: (0, 0))
    in_specs = [half, half, half, half]
    if affine:
        in_specs += [pl.BlockSpec((bn, 128), lambda i: (i, 0)),
                     pl.BlockSpec((bn, 128), lambda i: (i, 0)),
                     vec, vec, vec, vec]
    in_specs += [full, vec, full, vec]
    if pool:
        in_specs += [pl.BlockSpec((1, 1, bn), lambda i: (i, 0, 0))]
    out_specs = [half, half, vec, vec]
    out_shape = [jax.ShapeDtypeStruct((n, 128), jnp.float32),
                 jax.ShapeDtypeStruct((n, 128), jnp.float32),
                 jax.ShapeDtypeStruct((1, 256), jnp.float32),
                 jax.ShapeDtypeStruct((1, 256), jnp.float32)]
    if pool:
        out_specs += [pl.BlockSpec((_G, 256), lambda i: (0, 0)),
                      pl.BlockSpec((_G, 128), lambda i: (0, 0))]
        out_shape += [jax.ShapeDtypeStruct((_G, 256), jnp.float32),
                      jax.ShapeDtypeStruct((_G, 128), jnp.float32)]

    return pl.pallas_call(
        body, grid=(grid,), in_specs=in_specs, out_specs=out_specs,
        out_shape=out_shape)


@functools.lru_cache(maxsize=None)
def _make_head(n: int, out_dim: int):
    nf = float(n)

    def body(p, cnt, s_in, q_in, g_in, be_in, wf1, bf1, wf2, bf2, out):
        mu = s_in[...] * (1.0 / nf)
        var = q_in[...] * (1.0 / nf) - mu * mu
        a = g_in[...] * lax.rsqrt(var + _EPS)
        cbias = be_in[...] - mu * a
        pooled = a * p[...] + cbias * cnt[...][:, 0:1]
        t = jnp.maximum(
            jnp.dot(pooled, wf1[...], preferred_element_type=jnp.float32)
            + bf1[...], 0.0)
        logits = jnp.dot(t, wf2[...], preferred_element_type=jnp.float32) \
            + bf2[...]
        mx = jnp.max(logits, axis=-1, keepdims=True)
        z = logits - mx
        out[...] = z - jnp.log(jnp.sum(jnp.exp(z), axis=-1, keepdims=True))

    return pl.pallas_call(
        body, out_shape=jax.ShapeDtypeStruct((_G, out_dim), jnp.float32))


def kernel(x, edge_index, batch, W1a, b1a, W1b, b1b, g1, be1, W2a, b2a, W2b,
           b2b, g2, be2, W3a, b3a, W3b, b3b, g3, be3, Wf1, bf1, Wf2, bf2):
    n, d = x.shape
    e = edge_index.shape[1]
    out_dim = Wf2.shape[1]
    bn = 1000
    ept = e // _NS
    k = _chunk(ept, _AGGR_CAP)

    kd = _chunk(e // (_NC * _NS))
    xa, xb = x[:, :128], x[:, 128:]
    srcs = edge_index[0].reshape(_NS, ept // k, k)
    dsts = edge_index[1].reshape(_NS, ept // k, k)
    dstd = edge_index[1].reshape(_NC * _NS, e // (_NC * _NS) // kd, kd)
    bat3 = batch.reshape(n // bn, 1, bn)

    def r(v):
        return v.reshape(1, -1)

    deg0, deg1 = _deg_call(dstd, n)
    a1a, a1b = _aggr_call(xa, xb, srcs, dsts)
    y1a, y1b, s1, q1 = _make_layer(n, False, False, bn)(
        xa, xb, a1a, a1b, W1a, r(b1a), W1b, r(b1b))

    a2a, a2b = _aggr_call(y1a, y1b, srcs, dsts)
    y2a, y2b, s2, q2 = _make_layer(n, True, False, bn)(
        y1a, y1b, a2a, a2b, deg0, deg1, s1, q1, r(g1), r(be1),
        W2a, r(b2a), W2b, r(b2b))

    a3a, a3b = _aggr_call(y2a, y2b, srcs, dsts)
    _, _, s3, q3, p, cnt = _make_layer(n, True, True, bn)(
        y2a, y2b, a3a, a3b, deg0, deg1, s2, q2, r(g2), r(be2),
        W3a, r(b3a), W3b, r(b3b), bat3)

    return _make_head(n, out_dim)(
        p, cnt, s3, q3, r(g3), r(be3), Wf1, r(bf1), Wf2, r(bf2))


# speedup vs baseline: 1.2971x; 1.0052x over previous
import functools

import jax
import jax.numpy as jnp
from jax import lax
from jax.experimental import pallas as pl
from jax.experimental.pallas import tpu as pltpu
from jax.experimental.pallas import tpu_sc as plsc

_NC = 2
_NS = 16
_G = 64
_EPS = 1e-5


def _chunk(ept: int, cap: int = 128) -> int:
    for k in range(cap, 0, -8):
        if ept % k == 0:
            return k
    raise ValueError(ept)


@functools.lru_cache(maxsize=None)
def _make_deg(n: int, e: int):
    nw = _NC * _NS
    k = 128
    ept = e // nw
    nch = ept // k
    rpt = (n // (8 * _NS)) * 8
    tail = n - rpt * _NS
    mesh = plsc.VectorSubcoreMesh(
        core_axis_name="c", subcore_axis_name="s", num_cores=_NC)
    out_type = [jax.ShapeDtypeStruct((n, 128), jnp.float32)] * 2
    scratch = [
        pltpu.VMEM((nch, k), jnp.int32),
        pltpu.VMEM((k, 128), jnp.float32),
        pltpu.VMEM_SHARED((n + 8, 128), jnp.float32),
    ]

    @functools.partial(pl.kernel, mesh=mesh, out_type=out_type,
                       scratch_types=scratch)
    def deg(dsts, z128, o128, d0, d1, idst, ones_v, dacc):
        c = lax.axis_index("c")
        s = lax.axis_index("s")
        row0 = s * rpt
        pltpu.sync_copy(z128, dacc.at[pl.ds(row0, rpt)])
        if tail:
            @pl.when(s == _NS - 1)
            def _zt():
                pltpu.sync_copy(z128.at[pl.ds(0, tail)],
                                dacc.at[pl.ds(rpt * _NS, tail)])
        pltpu.sync_copy(o128, ones_v)
        pltpu.sync_copy(dsts.at[c * _NS + s], idst)
        plsc.subcore_barrier()

        def body(t, carry):
            pltpu.sync_copy(ones_v, dacc.at[idst.at[t]], add=True)
            return carry
        lax.fori_loop(0, nch, body, 0)
        plsc.subcore_barrier()

        sl = pl.ds(row0, rpt)
        tl = pl.ds(rpt * _NS, tail) if tail else None

        @pl.when(c == 0)
        def _o0():
            pltpu.sync_copy(dacc.at[sl], d0.at[sl])
            if tail:
                @pl.when(s == _NS - 1)
                def _o0t():
                    pltpu.sync_copy(dacc.at[tl], d0.at[tl])

        @pl.when(c == 1)
        def _o1():
            pltpu.sync_copy(dacc.at[sl], d1.at[sl])
            if tail:
                @pl.when(s == _NS - 1)
                def _o1t():
                    pltpu.sync_copy(dacc.at[tl], d1.at[tl])

    return deg


def _deg_call(dstd, n):
    e = dstd.shape[0] * dstd.shape[1] * dstd.shape[2]
    rpt = (n // (8 * _NS)) * 8
    k = dstd.shape[2]
    z128 = jnp.zeros((rpt, 128), jnp.float32)
    o128 = jnp.ones((k, 128), jnp.float32)
    return _make_deg(n, e)(dstd, z128, o128)


def _deg_plan(e):
    ept = -(-e // (_NC * _NS * 128)) * 128
    return ept, ept * _NC * _NS - e


@functools.lru_cache(maxsize=None)
def _make_aggr(n: int, e: int, cap: int):
    ept = e // _NS
    k = _chunk(ept, cap)
    nch = ept // k
    rpt = (n // (8 * _NS)) * 8
    tail = n - rpt * _NS
    mesh = plsc.VectorSubcoreMesh(
        core_axis_name="c", subcore_axis_name="s", num_cores=_NC)

    out_type = [
        jax.ShapeDtypeStruct((n, 128), jnp.float32),
        jax.ShapeDtypeStruct((n, 128), jnp.float32),
    ]
    scratch = [
        pltpu.VMEM((nch, k), jnp.int32),
        pltpu.VMEM((nch, k), jnp.int32),
        pltpu.VMEM((k, 128), jnp.float32),
        pltpu.VMEM_SHARED((n, 128), jnp.float32),
        pltpu.SemaphoreType.DMA,
    ]

    @functools.partial(pl.kernel, mesh=mesh, out_type=out_type,
                       scratch_types=scratch)
    def aggr(ya, yb, srcs, dsts, z128, aa, ab, isrc, idst, rows, acc, sem):
        c = lax.axis_index("c")
        s = lax.axis_index("s")
        row0 = s * rpt

        pltpu.sync_copy(z128, acc.at[pl.ds(row0, rpt)])
        if tail:
            @pl.when(s == _NS - 1)
            def _zero_tail():
                pltpu.sync_copy(z128.at[pl.ds(0, tail)],
                                acc.at[pl.ds(rpt * _NS, tail)])

        pltpu.sync_copy(srcs.at[s], isrc)
        pltpu.sync_copy(dsts.at[s], idst)
        plsc.subcore_barrier()

        def run(yref):
            def body(t, carry):
                pltpu.async_copy(yref.at[isrc.at[t]], rows, sem).wait()
                pltpu.sync_copy(rows, acc.at[idst.at[t]], add=True)
                return carry
            lax.fori_loop(0, nch, body, 0)

        @pl.when(c == 0)
        def _run_a():
            run(ya)

        @pl.when(c == 1)
        def _run_b():
            run(yb)

        plsc.subcore_barrier()

        sl = pl.ds(row0, rpt)
        tl = pl.ds(rpt * _NS, tail) if tail else None

        @pl.when(c == 0)
        def _out_a():
            pltpu.sync_copy(acc.at[sl], aa.at[sl])
            if tail:
                @pl.when(s == _NS - 1)
                def _out_a_tail():
                    pltpu.sync_copy(acc.at[tl], aa.at[tl])

        @pl.when(c == 1)
        def _out_b():
            pltpu.sync_copy(acc.at[sl], ab.at[sl])
            if tail:
                @pl.when(s == _NS - 1)
                def _out_b_tail():
                    pltpu.sync_copy(acc.at[tl], ab.at[tl])

    return aggr


_AGGR_CAP = 80


def _aggr_call(ya, yb, srcs, dsts):
    n = ya.shape[0]
    e = srcs.shape[0] * srcs.shape[1] * srcs.shape[2]
    rpt = (n // (8 * _NS)) * 8
    z128 = jnp.zeros((rpt, 128), jnp.float32)
    return _make_aggr(n, e, _AGGR_CAP)(ya, yb, srcs, dsts, z128)


@functools.lru_cache(maxsize=None)
def _make_layer(n: int, affine: bool, pool: bool, bn: int):
    grid = n // bn
    nf = float(n)

    def body(*refs):
        i = pl.program_id(0)
        it = iter(refs)
        ya, yb, aa, ab = next(it), next(it), next(it), next(it)
        if affine:
            d0, d1, s_in, q_in, g_in, be_in = (next(it), next(it), next(it),
                                               next(it), next(it), next(it))
        wa, ba, wb, bb = next(it), next(it), next(it), next(it)
        if pool:
            bat = next(it)
        oa, ob, s_o, q_o = next(it), next(it), next(it), next(it)
        if pool:
            p_o, cnt_o = next(it), next(it)

        y = jnp.concatenate([ya[...], yb[...]], axis=1)
        agg = jnp.concatenate([aa[...], ab[...]], axis=1)
        if affine:
            mu = s_in[...] * (1.0 / nf)
            var = q_in[...] * (1.0 / nf) - mu * mu
            a = g_in[...] * lax.rsqrt(var + _EPS)
            cbias = be_in[...] - mu * a
            m = a * (y + agg) + cbias * (
                1.0 + (d0[...] + d1[...])[:, 0:1])
        else:
            m = y + agg
        t = jnp.maximum(
            jnp.dot(m, wa[...], preferred_element_type=jnp.float32) + ba[...],
            0.0)
        y2 = jnp.maximum(
            jnp.dot(t, wb[...], preferred_element_type=jnp.float32) + bb[...],
            0.0)
        oa[...] = y2[:, :128]
        ob[...] = y2[:, 128:]
        s_p = jnp.sum(y2, axis=0, keepdims=True)
        q_p = jnp.sum(y2 * y2, axis=0, keepdims=True)
        if pool:
            iota_g = lax.broadcasted_iota(jnp.int32, (_G, bn), 0)
            onehot = (jnp.reshape(bat[...], (1, bn)) == iota_g
                      ).astype(jnp.float32)
            p_p = jnp.dot(onehot, y2, preferred_element_type=jnp.float32)
            c_p = jnp.sum(onehot, axis=1, keepdims=True) + jnp.zeros(
                (_G, 128), jnp.float32)

        @pl.when(i == 0)
        def _init():
            s_o[...] = s_p
            q_o[...] = q_p
            if pool:
                p_o[...] = p_p
                cnt_o[...] = c_p

        @pl.when(i > 0)
        def _accum():
            s_o[...] += s_p
            q_o[...] += q_p
            if pool:
                p_o[...] += p_p
                cnt_o[...] += c_p

    half = pl.BlockSpec((bn, 128), lambda i: (i, 0))
    full = pl.BlockSpec((256, 256), lambda i: (0, 0))
    vec = pl.BlockSpec((1, 256), lambda i: (0, 0))
    in_specs = [half, half, half, half]
    if affine:
        in_specs += [pl.BlockSpec((bn, 128), lambda i: (i, 0)),
                     pl.BlockSpec((bn, 128), lambda i: (i, 0)),
                     vec, vec, vec, vec]
    in_specs += [full, vec, full, vec]
    if pool:
        in_specs += [pl.BlockSpec((1, 1, bn), lambda i: (i, 0, 0))]
    out_specs = [half, half, vec, vec]
    out_shape = [jax.ShapeDtypeStruct((n, 128), jnp.float32),
                 jax.ShapeDtypeStruct((n, 128), jnp.float32),
                 jax.ShapeDtypeStruct((1, 256), jnp.float32),
                 jax.ShapeDtypeStruct((1, 256), jnp.float32)]
    if pool:
        out_specs += [pl.BlockSpec((_G, 256), lambda i: (0, 0)),
                      pl.BlockSpec((_G, 128), lambda i: (0, 0))]
        out_shape += [jax.ShapeDtypeStruct((_G, 256), jnp.float32),
                      jax.ShapeDtypeStruct((_G, 128), jnp.float32)]

    return pl.pallas_call(
        body, grid=(grid,), in_specs=in_specs, out_specs=out_specs,
        out_shape=out_shape)


@functools.lru_cache(maxsize=None)
def _make_head(n: int, out_dim: int):
    nf = float(n)

    def body(p, cnt, s_in, q_in, g_in, be_in, wf1, bf1, wf2, bf2, out):
        mu = s_in[...] * (1.0 / nf)
        var = q_in[...] * (1.0 / nf) - mu * mu
        a = g_in[...] * lax.rsqrt(var + _EPS)
        cbias = be_in[...] - mu * a
        pooled = a * p[...] + cbias * cnt[...][:, 0:1]
        t = jnp.maximum(
            jnp.dot(pooled, wf1[...], preferred_element_type=jnp.float32)
            + bf1[...], 0.0)
        logits = jnp.dot(t, wf2[...], preferred_element_type=jnp.float32) \
            + bf2[...]
        mx = jnp.max(logits, axis=-1, keepdims=True)
        z = logits - mx
        out[...] = z - jnp.log(jnp.sum(jnp.exp(z), axis=-1, keepdims=True))

    return pl.pallas_call(
        body, out_shape=jax.ShapeDtypeStruct((_G, out_dim), jnp.float32))


def kernel(x, edge_index, batch, W1a, b1a, W1b, b1b, g1, be1, W2a, b2a, W2b,
           b2b, g2, be2, W3a, b3a, W3b, b3b, g3, be3, Wf1, bf1, Wf2, bf2):
    n, d = x.shape
    e = edge_index.shape[1]
    out_dim = Wf2.shape[1]
    bn = 1000
    ept = e // _NS
    k = _chunk(ept, _AGGR_CAP)

    eptd, padd = _deg_plan(e)
    xa, xb = x[:, :128], x[:, 128:]
    srcs = edge_index[0].reshape(_NS, ept // k, k)
    dsts = edge_index[1].reshape(_NS, ept // k, k)
    dstd = jnp.concatenate(
        [edge_index[1], jnp.full((padd,), n, jnp.int32)]
    ).reshape(_NC * _NS, eptd // 128, 128)
    bat3 = batch.reshape(n // bn, 1, bn)

    def r(v):
        return v.reshape(1, -1)

    deg0, deg1 = _deg_call(dstd, n)
    a1a, a1b = _aggr_call(xa, xb, srcs, dsts)
    y1a, y1b, s1, q1 = _make_layer(n, False, False, bn)(
        xa, xb, a1a, a1b, W1a, r(b1a), W1b, r(b1b))

    a2a, a2b = _aggr_call(y1a, y1b, srcs, dsts)
    y2a, y2b, s2, q2 = _make_layer(n, True, False, bn)(
        y1a, y1b, a2a, a2b, deg0, deg1, s1, q1, r(g1), r(be1),
        W2a, r(b2a), W2b, r(b2b))

    a3a, a3b = _aggr_call(y2a, y2b, srcs, dsts)
    _, _, s3, q3, p, cnt = _make_layer(n, True, True, bn)(
        y2a, y2b, a3a, a3b, deg0, deg1, s2, q2, r(g2), r(be2),
        W3a, r(b3a), W3b, r(b3b), bat3)

    return _make_head(n, out_dim)(
        p, cnt, s3, q3, r(g3), r(be3), Wf1, r(bf1), Wf2, r(bf2))
